# baseline (speedup 1.0000x reference)
import functools

import jax
import jax.numpy as jnp
from jax import lax
from jax.experimental import pallas as pl
from jax.experimental.pallas import tpu as pltpu
from jax.experimental.pallas import tpu_sc as plsc

NB2 = 8


def _logits(w, zb, srow, crow, cb_ref):
    m = lax.dot_general(zb, cb_ref[...], (((0,), (1,)), ((), ())),
                        preferred_element_type=jnp.float32)
    dist = (srow + crow) - 2.0 * m
    return -(w * dist)


def _idx_body(w_ref, z_ref, s_ref, c_ref, cb_ref, idx_ref, rev_s, *, size):
    step = pl.program_id(0)
    w = w_ref[0, 0]

    @pl.when(step == 0)
    def _prep():
        rev_s[...] = (jnp.int32(size) - lax.broadcasted_iota(
            jnp.int32, (1, size), 1)).astype(jnp.float32)

    rev = rev_s[...]
    crow = c_ref[...]
    for i in range(NB2):
        g = _logits(w, z_ref[i], s_ref[i], crow, cb_ref)
        rowmax = jnp.max(g, axis=1)
        vmax = jnp.max(jnp.where(g == rowmax[:, None], rev, 0.0), axis=1)
        idx_ref[i, 0] = jnp.int32(size) - vmax.astype(jnp.int32)


def _loss_body(w_ref, z_ref, s_ref, c_ref, cb_ref, loss_ref, *, inv_bs):
    step = pl.program_id(0)
    w = w_ref[0, 0]
    crow = c_ref[...]
    loss = jnp.float32(0.0)
    for i in range(NB2):
        g = _logits(w, z_ref[i], s_ref[i], crow, cb_ref)
        rowmax = jnp.max(g, axis=1)
        e = jnp.exp(g - rowmax[:, None])
        denom = jnp.sum(e, axis=1)
        sumeg = jnp.sum(e * g, axis=1)
        kld = jnp.sum(sumeg / denom - rowmax - jnp.log(denom))
        loss += kld - jnp.sum(rowmax)

    @pl.when(step == 0)
    def _init():
        loss_ref[0, 0] = 0.0

    loss_ref[0, 0] += loss * inv_bs


def _sc_gather_body(table_hbm, idx_hbm, out_hbm, idx_v, rows_v, sem, *,
                    chunk, nchunks, nc):
    wid = lax.axis_index("s") * nc + lax.axis_index("c")
    for j in range(nchunks):
        c = wid * nchunks + j
        pltpu.sync_copy(idx_hbm.at[pl.ds(c * chunk, chunk)], idx_v)
        pltpu.async_copy(table_hbm.at[idx_v], rows_v, sem).wait()
        pltpu.sync_copy(rows_v, out_hbm.at[pl.ds(c * chunk, chunk)])


def kernel(z, codebook, var_q, var_init):
    bs, dim_z, d1, d2 = z.shape
    size, _ = codebook.shape
    npix = d1 * d2
    ntok = bs * npix
    z3 = z.reshape(bs, dim_z, npix)

    var_q_eff = jax.nn.sigmoid(var_q) * 2.0 * var_init
    w = (0.5 / jnp.clip(var_q_eff, 1e-10, None)).reshape(1, 1)

    zf = jnp.transpose(z, (0, 2, 3, 1)).reshape(-1, dim_z)
    s = jnp.sum(zf ** 2, axis=1).reshape(bs, npix, 1)
    c = jnp.sum(codebook ** 2, axis=1).reshape(1, size)

    in_specs = [
        pl.BlockSpec(memory_space=pltpu.SMEM),
        pl.BlockSpec((NB2, dim_z, npix), lambda t: (t, 0, 0)),
        pl.BlockSpec((NB2, npix, 1), lambda t: (t, 0, 0)),
        pl.BlockSpec((1, size), lambda t: (0, 0)),
        pl.BlockSpec((size, dim_z), lambda t: (0, 0)),
    ]
    idx = pl.pallas_call(
        functools.partial(_idx_body, size=size),
        grid=(bs // NB2,),
        in_specs=in_specs,
        out_specs=pl.BlockSpec((NB2, 1, npix), lambda t: (t, 0, 0)),
        out_shape=jax.ShapeDtypeStruct((bs, 1, npix), jnp.int32),
        scratch_shapes=[pltpu.VMEM((1, size), jnp.float32)],
    )(w, z3, s, c, codebook)

    info = plsc.get_sparse_core_info()
    nc, ns = info.num_cores, info.num_subcores
    chunk = 512
    nchunks = ntok // (nc * ns * chunk)
    table = jnp.pad(codebook, ((0, 0), (0, 128 - dim_z)))
    mesh = plsc.VectorSubcoreMesh(core_axis_name="c", subcore_axis_name="s")
    sc_gather = pl.kernel(
        functools.partial(_sc_gather_body, chunk=chunk, nchunks=nchunks,
                          nc=nc),
        out_type=jax.ShapeDtypeStruct((ntok, 128), jnp.float32),
        mesh=mesh,
        scratch_types=[
            pltpu.VMEM((chunk,), jnp.int32),
            pltpu.VMEM((chunk, 128), jnp.float32),
            pltpu.SemaphoreType.DMA,
        ],
    )
    rows = sc_gather(table, idx.reshape(ntok))

    loss = pl.pallas_call(
        functools.partial(_loss_body, inv_bs=1.0 / bs),
        grid=(bs // NB2,),
        in_specs=in_specs,
        out_specs=pl.BlockSpec(memory_space=pltpu.SMEM),
        out_shape=jax.ShapeDtypeStruct((1, 1), jnp.float32),
    )(w, z3, s, c, codebook)

    zq = rows[:, :dim_z].reshape(bs, npix, dim_z).transpose(0, 2, 1)
    return zq.reshape(bs, dim_z, d1, d2), loss[0, 0]


# speedup vs baseline: 1.0720x; 1.0720x over previous
import functools

import jax
import jax.numpy as jnp
from jax import lax
from jax.experimental import pallas as pl
from jax.experimental.pallas import tpu as pltpu
from jax.experimental.pallas import tpu_sc as plsc

NB2 = 8


def _vq_body(w_ref, z_ref, s_ref, c_ref, cb_ref, idx_ref, loss_ref, rev_s, *,
             size, dim, npix, inv_bs):
    step = pl.program_id(0)
    w = w_ref[0, 0]

    @pl.when(step == 0)
    def _prep():
        rev_s[...] = (jnp.int32(size) - lax.broadcasted_iota(
            jnp.int32, (1, size), 1)).astype(jnp.float32)

    rev = rev_s[...]
    crow = c_ref[...]
    loss = jnp.float32(0.0)
    for i in range(NB2):
        zb = z_ref[i]
        m = lax.dot_general(zb, cb_ref[...], (((0,), (1,)), ((), ())),
                            preferred_element_type=jnp.float32)
        dist = (s_ref[i] + crow) - 2.0 * m
        g = -(w * dist)

        rowmax = jnp.max(g, axis=1)
        e = jnp.exp(g - rowmax[:, None])
        denom = jnp.sum(e, axis=1)
        sumeg = jnp.sum(e * g, axis=1)
        kld = jnp.sum(sumeg / denom - rowmax - jnp.log(denom))

        vmax = jnp.max(jnp.where(g == rowmax[:, None], rev, 0.0), axis=1)
        idx_ref[i, 0] = jnp.int32(size) - vmax.astype(jnp.int32)

        loss += kld - jnp.sum(rowmax)

    @pl.when(step == 0)
    def _init():
        loss_ref[0, 0] = 0.0

    loss_ref[0, 0] += loss * inv_bs


def _sc_gather_body(table_hbm, idx_hbm, out_hbm, idx_v, rows_v, sem, *,
                    chunk, nchunks, nc):
    wid = lax.axis_index("s") * nc + lax.axis_index("c")
    for j in range(nchunks):
        c = wid * nchunks + j
        pltpu.sync_copy(idx_hbm.at[pl.ds(c * chunk, chunk)], idx_v)
        pltpu.async_copy(table_hbm.at[idx_v], rows_v, sem).wait()
        pltpu.sync_copy(rows_v, out_hbm.at[pl.ds(c * chunk, chunk)])


def kernel(z, codebook, var_q, var_init):
    bs, dim_z, d1, d2 = z.shape
    size, _ = codebook.shape
    npix = d1 * d2
    ntok = bs * npix
    z3 = z.reshape(bs, dim_z, npix)

    var_q_eff = jax.nn.sigmoid(var_q) * 2.0 * var_init
    w = (0.5 / jnp.clip(var_q_eff, 1e-10, None)).reshape(1, 1)

    zf = jnp.transpose(z, (0, 2, 3, 1)).reshape(-1, dim_z)
    s = jnp.sum(zf ** 2, axis=1).reshape(bs, npix, 1)
    c = jnp.sum(codebook ** 2, axis=1).reshape(1, size)

    body = functools.partial(_vq_body, size=size, dim=dim_z, npix=npix,
                             inv_bs=1.0 / bs)
    idx, loss = pl.pallas_call(
        body,
        grid=(bs // NB2,),
        in_specs=[
            pl.BlockSpec(memory_space=pltpu.SMEM),
            pl.BlockSpec((NB2, dim_z, npix), lambda t: (t, 0, 0)),
            pl.BlockSpec((NB2, npix, 1), lambda t: (t, 0, 0)),
            pl.BlockSpec((1, size), lambda t: (0, 0)),
            pl.BlockSpec((size, dim_z), lambda t: (0, 0)),
        ],
        out_specs=[
            pl.BlockSpec((NB2, 1, npix), lambda t: (t, 0, 0)),
            pl.BlockSpec(memory_space=pltpu.SMEM),
        ],
        out_shape=[
            jax.ShapeDtypeStruct((bs, 1, npix), jnp.int32),
            jax.ShapeDtypeStruct((1, 1), jnp.float32),
        ],
        scratch_shapes=[
            pltpu.VMEM((1, size), jnp.float32),
        ],
    )(w, z3, s, c, codebook)

    info = plsc.get_sparse_core_info()
    nc, ns = info.num_cores, info.num_subcores
    chunk = 512
    nchunks = ntok // (nc * ns * chunk)
    table = jnp.pad(codebook, ((0, 0), (0, 128 - dim_z)))
    mesh = plsc.VectorSubcoreMesh(core_axis_name="c", subcore_axis_name="s")
    sc_gather = pl.kernel(
        functools.partial(_sc_gather_body, chunk=chunk, nchunks=nchunks,
                          nc=nc),
        out_type=jax.ShapeDtypeStruct((ntok, 128), jnp.float32),
        mesh=mesh,
        scratch_types=[
            pltpu.VMEM((chunk,), jnp.int32),
            pltpu.VMEM((chunk, 128), jnp.float32),
            pltpu.SemaphoreType.DMA,
        ],
    )
    rows = sc_gather(table, idx.reshape(ntok))

    zq = rows[:, :dim_z].reshape(bs, npix, dim_z).transpose(0, 2, 1)
    return zq.reshape(bs, dim_z, d1, d2), loss[0, 0]
